# baseline (speedup 1.0000x reference)
import functools

import jax
import jax.numpy as jnp
from jax import lax
from jax.experimental import pallas as pl
from jax.experimental.pallas import tpu as pltpu
from jax.experimental.pallas import tpu_sc as plsc

N = 1048576
TOT = 2 * N
NC = 2
NS = 16
NW = NC * NS
CHUNK = TOT // NW
K0 = N // 2 - 1
K1 = N // 2

H1 = 4096
H2 = 8192
H3 = 4096

M31 = 0x7FFFFFFF

_mesh = plsc.VectorSubcoreMesh(core_axis_name="c", subcore_axis_name="s")
_cparams = pltpu.CompilerParams(needs_layout_passes=False)


def _ids():
    cid = lax.axis_index("c")
    sid = lax.axis_index("s")
    return cid, sid, sid * NC + cid


def _key16(v):
    u = plsc.bitcast(v, jnp.int32)
    return u ^ ((u >> 31) & M31)


def _lane_iota():
    return lax.iota(jnp.int32, 16)


def _pick(vec, i):
    return jnp.sum(jnp.where(_lane_iota() == i, vec, jnp.int32(0)))


def _zero_flat(ref, n):
    z = jnp.zeros((16,), jnp.int32)

    @plsc.parallel_loop(0, n // 16)
    def _(j):
        ref[pl.ds(j * 16, 16)] = z


def _cum_select2(acc, base, nb, ka, kb):
    def body(i, carry):
        run, bca, rsa, bcb, rsb = carry
        s = pl.ds(base + i * 16, 16)
        v = acc[0, s] + acc[1, s]
        cs = plsc.cumsum(v) + run
        ma = cs <= ka
        mb = cs <= kb
        one = jnp.int32(1)
        zero = jnp.int32(0)
        bca = bca + jnp.where(ma, one, zero)
        rsa = rsa + jnp.where(ma, v, zero)
        bcb = bcb + jnp.where(mb, one, zero)
        rsb = rsb + jnp.where(mb, v, zero)
        run = run + jnp.sum(v)
        return run, bca, rsa, bcb, rsb

    z = jnp.zeros((16,), jnp.int32)
    _, bca, rsa, bcb, rsb = lax.fori_loop(
        0, nb // 16, body, (jnp.int32(0), z, z, z, z)
    )
    return jnp.sum(bca), ka - jnp.sum(rsa), jnp.sum(bcb), kb - jnp.sum(rsb)


def _pass_epilogue(out, hist, shared, tmp, red, sem, h):
    cid, sid, _ = _ids()
    w = h // 16
    pltpu.sync_copy(hist, shared.at[sid])
    plsc.subcore_barrier()
    base = sid * w
    pltpu.sync_copy(shared.at[:, pl.ds(base, w)], tmp)

    @plsc.parallel_loop(0, w // 16)
    def _(j):
        s = pl.ds(j * 16, 16)
        acc = tmp[0, s]
        for r in range(1, 16):
            acc = acc + tmp[r, s]
        red[s] = acc

    pltpu.sync_copy(red, out.at[cid, pl.ds(base, w)])


def _pass_scratch(h):
    return [
        pltpu.VMEM((CHUNK,), jnp.float32),
        pltpu.VMEM((h,), jnp.int32),
        pltpu.VMEM_SHARED((16, h), jnp.int32),
        pltpu.VMEM((16, h // 16), jnp.int32),
        pltpu.VMEM((h // 16,), jnp.int32),
        pltpu.SemaphoreType.DMA,
    ]


def _write_selvec(outsv, svm, pref, resid):
    _, _, wid = _ids()

    @pl.when(wid == 0)
    def _():
        lane = _lane_iota()
        par = lane & 1
        svm[0] = jnp.where(par == 0, pref[0][0], pref[0][1])
        svm[1] = jnp.where(par == 0, pref[1][0], pref[1][1])
        resids = jnp.zeros((16,), jnp.int32)
        for c in range(2):
            for k in range(2):
                resids = jnp.where(lane == 2 * c + k, resid[k][c], resids)
        svm[2] = resids
        pltpu.sync_copy(svm, outsv)


@functools.partial(
    pl.kernel,
    out_type=jax.ShapeDtypeStruct((NC, H1), jnp.int32),
    mesh=_mesh,
    compiler_params=_cparams,
    scratch_types=[
        pltpu.VMEM((2, CHUNK // 2), jnp.float32),
        pltpu.VMEM((H1,), jnp.int32),
        pltpu.VMEM_SHARED((16, H1), jnp.int32),
        pltpu.VMEM((16, H1 // 16), jnp.int32),
        pltpu.VMEM((H1 // 16,), jnp.int32),
        pltpu.SemaphoreType.DMA,
        pltpu.SemaphoreType.DMA,
    ],
)
def _pass1(data, out, buf, hist, shared, tmp, red, sem, sem2):
    _, _, wid = _ids()
    half = CHUNK // 2
    cp0 = pltpu.async_copy(data.at[pl.ds(wid * CHUNK, half)], buf.at[0], sem)
    cp1 = pltpu.async_copy(
        data.at[pl.ds(wid * CHUNK + half, half)], buf.at[1], sem2
    )
    _zero_flat(hist, H1)
    ones = jnp.ones((16,), jnp.int32)
    for hb, cp in ((0, cp0), (1, cp1)):
        cp.wait()

        @plsc.parallel_loop(0, half // 64, unroll=4)
        def _(t, hb=hb):
            base = t * 64
            coff = ((t >> 1) & 1) * 2048 + 1024
            for jj in range(4):
                v = buf[hb, pl.ds(base + jj * 16, 16)]
                key = _key16(v)
                idx = (key >> 21) + coff
                plsc.addupdate_scatter(hist, [idx], ones)

    _pass_epilogue(out, hist, shared, tmp, red, sem, H1)


@functools.partial(
    pl.kernel,
    out_type=(
        jax.ShapeDtypeStruct((NC, H2), jnp.int32),
        jax.ShapeDtypeStruct((3, 16), jnp.int32),
    ),
    mesh=_mesh,
    compiler_params=_cparams,
    scratch_types=_pass_scratch(H2) + [
        pltpu.VMEM((NC, H1), jnp.int32),
        pltpu.VMEM((3, 16), jnp.int32),
    ],
)
def _pass2(data, h1, outh, outsv, buf, hist, shared, tmp, red, sem, accv, svm):
    _, _, wid = _ids()
    cp = pltpu.async_copy(data.at[pl.ds(wid * CHUNK, CHUNK)], buf, sem)
    pltpu.sync_copy(h1, accv)
    _zero_flat(hist, H2)
    pref = [[None, None], [None, None]]
    resid = [[None, None], [None, None]]
    for c in range(2):
        b0, r0, b1, r1 = _cum_select2(
            accv, c * 2048, 2048, jnp.int32(K0), jnp.int32(K1)
        )
        pref[0][c], resid[0][c] = b0, r0
        pref[1][c], resid[1][c] = b1, r1
    eq = jnp.logical_and(pref[0][0] == pref[1][0], pref[0][1] == pref[1][1])
    cp.wait()
    ones = jnp.ones((16,), jnp.int32)

    prefv = [
        [jnp.broadcast_to(pref[k][c], (16,)) for c in range(2)]
        for k in range(2)
    ]

    def scan(dual):
        @plsc.parallel_loop(0, CHUNK // 64, unroll=4)
        def _(t):
            base = t * 64
            cc = (t >> 1) & 1
            p0 = jnp.where(cc == 0, prefv[0][0], prefv[0][1])
            p1 = jnp.where(cc == 0, prefv[1][0], prefv[1][1])
            coff = cc * 2048
            for jj in range(4):
                v = buf[pl.ds(base + jj * 16, 16)]
                key = _key16(v)
                b1v = (key >> 21) + 1024
                idx = ((key >> 10) & 0x7FF) + coff
                plsc.addupdate_scatter(hist, [idx], ones, mask=b1v == p0)
                if dual:
                    plsc.addupdate_scatter(
                        hist, [idx + 4096], ones, mask=b1v == p1
                    )

    @pl.when(eq)
    def _():
        scan(False)

    @pl.when(jnp.logical_not(eq))
    def _():
        scan(True)

    _pass_epilogue(outh, hist, shared, tmp, red, sem, H2)
    _write_selvec(outsv, svm, pref, resid)


@functools.partial(
    pl.kernel,
    out_type=(
        jax.ShapeDtypeStruct((NC, H3), jnp.int32),
        jax.ShapeDtypeStruct((3, 16), jnp.int32),
    ),
    mesh=_mesh,
    compiler_params=_cparams,
    scratch_types=_pass_scratch(H3) + [
        pltpu.VMEM((NC, H2), jnp.int32),
        pltpu.VMEM((3, 16), jnp.int32),
        pltpu.VMEM((3, 16), jnp.int32),
    ],
)
def _pass3(data, h2, sv1, outh, outsv, buf, hist, shared, tmp, red, sem,
           accv, svin, svm):
    _, _, wid = _ids()
    cp = pltpu.async_copy(data.at[pl.ds(wid * CHUNK, CHUNK)], buf, sem)
    pltpu.sync_copy(h2, accv)
    pltpu.sync_copy(sv1, svin)
    _zero_flat(hist, H3)
    b1s = [[_pick(svin[k], c) for c in range(2)] for k in range(2)]
    rins = [[_pick(svin[2], 2 * c + k) for c in range(2)] for k in range(2)]
    eq2 = jnp.logical_and(b1s[0][0] == b1s[1][0], b1s[0][1] == b1s[1][1])
    pref = [[None, None], [None, None]]
    resid = [[None, None], [None, None]]
    for k in range(2):
        for c in range(2):
            kbase = k * 4096 if k == 0 else jnp.where(eq2, 0, 4096)
            b2, r2, _, _ = _cum_select2(
                accv, kbase + c * 2048, 2048, rins[k][c], rins[k][c]
            )
            pref[k][c] = (((b1s[k][c] + 1024) & 0x7FF) << 11) | b2
            resid[k][c] = r2
    eq3 = jnp.logical_and(pref[0][0] == pref[1][0], pref[0][1] == pref[1][1])
    cp.wait()
    ones = jnp.ones((16,), jnp.int32)

    prefv = [
        [jnp.broadcast_to(pref[k][c], (16,)) for c in range(2)]
        for k in range(2)
    ]

    def scan(dual):
        @plsc.parallel_loop(0, CHUNK // 64, unroll=4)
        def _(t):
            base = t * 64
            cc = (t >> 1) & 1
            p0 = jnp.where(cc == 0, prefv[0][0], prefv[0][1])
            p1 = jnp.where(cc == 0, prefv[1][0], prefv[1][1])
            coff = cc * 1024
            for jj in range(4):
                v = buf[pl.ds(base + jj * 16, 16)]
                key = _key16(v)
                p22 = (key >> 10) & 0x3FFFFF
                idx = (key & 0x3FF) + coff
                plsc.addupdate_scatter(hist, [idx], ones, mask=p22 == p0)
                if dual:
                    plsc.addupdate_scatter(
                        hist, [idx + 2048], ones, mask=p22 == p1
                    )

    @pl.when(eq3)
    def _():
        scan(False)

    @pl.when(jnp.logical_not(eq3))
    def _():
        scan(True)

    _pass_epilogue(outh, hist, shared, tmp, red, sem, H3)
    _write_selvec(outsv, svm, pref, resid)


@functools.partial(
    pl.kernel,
    out_type=jax.ShapeDtypeStruct((16,), jnp.float32),
    mesh=_mesh,
    compiler_params=_cparams,
    scratch_types=[
        pltpu.VMEM((NC, H3), jnp.int32),
        pltpu.VMEM((3, 16), jnp.int32),
        pltpu.VMEM((16,), jnp.float32),
    ],
)
def _sel3(hin, sel, out, acc, svin, ovm):
    _, _, wid = _ids()

    @pl.when(wid == 0)
    def _():
        pltpu.sync_copy(hin, acc)
        pltpu.sync_copy(sel, svin)
        lane = _lane_iota()
        p22s = [[_pick(svin[k], c) for c in range(2)] for k in range(2)]
        rins = [[_pick(svin[2], 2 * c + k) for c in range(2)] for k in range(2)]
        eq3 = jnp.logical_and(
            p22s[0][0] == p22s[1][0], p22s[0][1] == p22s[1][1]
        )
        meds = []
        for c in range(2):
            vals = []
            for k in range(2):
                kbase = k * 2048 if k == 0 else jnp.where(eq3, 0, 2048)
                b3, _, _, _ = _cum_select2(
                    acc, kbase + c * 1024, 1024, rins[k][c], rins[k][c]
                )
                key = (p22s[k][c] << 10) | b3
                u = key ^ ((key >> 31) & M31)
                vals.append(lax.bitcast_convert_type(u, jnp.float32))
            meds.append((vals[0] + vals[1]) * jnp.float32(0.5))
        zv = jnp.zeros((16,), jnp.float32)
        res = jnp.where(lane == 0, meds[0], zv)
        res = jnp.where(lane == 1, meds[1], res)
        ovm[pl.ds(0, 16)] = res
        pltpu.sync_copy(ovm, out)


def kernel(inputs):
    flat = inputs.reshape(8192, 128, 2).transpose(0, 2, 1).reshape(-1)
    h1 = _pass1(flat)
    h2, sv1 = _pass2(flat, h1)
    h3, sv2 = _pass3(flat, h2, sv1)
    med = _sel3(h3, sv2)
    return med[:2]


# speedup vs baseline: 1.0070x; 1.0070x over previous
import functools

import jax
import jax.numpy as jnp
from jax import lax
from jax.experimental import pallas as pl
from jax.experimental.pallas import tpu as pltpu
from jax.experimental.pallas import tpu_sc as plsc

N = 1048576
TOT = 2 * N
NC = 2
NS = 16
NW = NC * NS
CHUNK = TOT // NW
K0 = N // 2 - 1
K1 = N // 2

H1 = 4096
H2 = 8192
H3 = 4096

M31 = 0x7FFFFFFF

_mesh = plsc.VectorSubcoreMesh(core_axis_name="c", subcore_axis_name="s")
_cparams = pltpu.CompilerParams(needs_layout_passes=False)


def _ids():
    cid = lax.axis_index("c")
    sid = lax.axis_index("s")
    return cid, sid, sid * NC + cid


def _key16(v):
    u = plsc.bitcast(v, jnp.int32)
    return u ^ ((u >> 31) & M31)


def _lane_iota():
    return lax.iota(jnp.int32, 16)


def _pick(vec, i):
    return jnp.sum(jnp.where(_lane_iota() == i, vec, jnp.int32(0)))


def _zero_flat(ref, n):
    z = jnp.zeros((16,), jnp.int32)

    @plsc.parallel_loop(0, n // 16)
    def _(j):
        ref[pl.ds(j * 16, 16)] = z


def _cum_select2(acc, bsum, base, nb, ka, kb):
    m = nb // 16

    lane = _lane_iota()

    @plsc.parallel_loop(0, m // 16)
    def _(g):
        sv = jnp.zeros((16,), jnp.int32)
        for l in range(16):
            s = pl.ds(base + (g * 16 + l) * 16, 16)
            sv = jnp.where(lane == l, jnp.sum(acc[0, s] + acc[1, s]), sv)
        bsum[pl.ds(g * 16, 16)] = sv

    def body(i, carry):
        run, bca, rsa, bcb, rsb = carry
        v = bsum[pl.ds(i * 16, 16)]
        cs = plsc.cumsum(v) + run
        ma = cs <= ka
        mb = cs <= kb
        one = jnp.int32(1)
        zero = jnp.int32(0)
        bca = bca + jnp.where(ma, one, zero)
        rsa = rsa + jnp.where(ma, v, zero)
        bcb = bcb + jnp.where(mb, one, zero)
        rsb = rsb + jnp.where(mb, v, zero)
        run = run + jnp.sum(v)
        return run, bca, rsa, bcb, rsb

    z = jnp.zeros((16,), jnp.int32)
    _, bca, rsa, bcb, rsb = lax.fori_loop(
        0, m // 16, body, (jnp.int32(0), z, z, z, z)
    )

    def descend(jb, r):
        s = pl.ds(base + jb * 16, 16)
        v = acc[0, s] + acc[1, s]
        cs = plsc.cumsum(v)
        mk = cs <= r
        bi = jnp.sum(jnp.where(mk, jnp.int32(1), jnp.int32(0)))
        resid = r - jnp.sum(jnp.where(mk, v, jnp.int32(0)))
        return jb * 16 + bi, resid

    ba, ra = descend(jnp.sum(bca), ka - jnp.sum(rsa))
    bb, rb = descend(jnp.sum(bcb), kb - jnp.sum(rsb))
    return ba, ra, bb, rb


def _pass_epilogue(out, hist, shared, tmp, red, sem, h):
    cid, sid, _ = _ids()
    w = h // 16
    pltpu.sync_copy(hist, shared.at[sid])
    plsc.subcore_barrier()
    base = sid * w
    pltpu.sync_copy(shared.at[:, pl.ds(base, w)], tmp)

    @plsc.parallel_loop(0, w // 16)
    def _(j):
        s = pl.ds(j * 16, 16)
        acc = tmp[0, s]
        for r in range(1, 16):
            acc = acc + tmp[r, s]
        red[s] = acc

    pltpu.sync_copy(red, out.at[cid, pl.ds(base, w)])


def _pass_scratch(h):
    return [
        pltpu.VMEM((CHUNK,), jnp.float32),
        pltpu.VMEM((h,), jnp.int32),
        pltpu.VMEM_SHARED((16, h), jnp.int32),
        pltpu.VMEM((16, h // 16), jnp.int32),
        pltpu.VMEM((h // 16,), jnp.int32),
        pltpu.SemaphoreType.DMA,
    ]


def _write_selvec(outsv, svm, pref, resid):
    _, _, wid = _ids()

    @pl.when(wid == 0)
    def _():
        lane = _lane_iota()
        par = lane & 1
        svm[0] = jnp.where(par == 0, pref[0][0], pref[0][1])
        svm[1] = jnp.where(par == 0, pref[1][0], pref[1][1])
        resids = jnp.zeros((16,), jnp.int32)
        for c in range(2):
            for k in range(2):
                resids = jnp.where(lane == 2 * c + k, resid[k][c], resids)
        svm[2] = resids
        pltpu.sync_copy(svm, outsv)


@functools.partial(
    pl.kernel,
    out_type=jax.ShapeDtypeStruct((NC, H1), jnp.int32),
    mesh=_mesh,
    compiler_params=_cparams,
    scratch_types=_pass_scratch(H1),
)
def _pass1(data, out, buf, hist, shared, tmp, red, sem):
    _, _, wid = _ids()
    cp = pltpu.async_copy(data.at[pl.ds(wid * CHUNK, CHUNK)], buf, sem)
    _zero_flat(hist, H1)
    cp.wait()
    ones = jnp.ones((16,), jnp.int32)

    @plsc.parallel_loop(0, CHUNK // 64, unroll=4)
    def _(t):
        base = t * 64
        coff = ((t >> 1) & 1) * 2048 + 1024
        for jj in range(4):
            v = buf[pl.ds(base + jj * 16, 16)]
            key = _key16(v)
            idx = (key >> 21) + coff
            plsc.addupdate_scatter(hist, [idx], ones)

    _pass_epilogue(out, hist, shared, tmp, red, sem, H1)


@functools.partial(
    pl.kernel,
    out_type=(
        jax.ShapeDtypeStruct((NC, H2), jnp.int32),
        jax.ShapeDtypeStruct((3, 16), jnp.int32),
    ),
    mesh=_mesh,
    compiler_params=_cparams,
    scratch_types=_pass_scratch(H2) + [
        pltpu.VMEM((NC, H1), jnp.int32),
        pltpu.VMEM((3, 16), jnp.int32),
        pltpu.VMEM((128,), jnp.int32),
    ],
)
def _pass2(data, h1, outh, outsv, buf, hist, shared, tmp, red, sem, accv, svm,
           bsum):
    _, _, wid = _ids()
    cp = pltpu.async_copy(data.at[pl.ds(wid * CHUNK, CHUNK)], buf, sem)
    pltpu.sync_copy(h1, accv)
    _zero_flat(hist, H2)
    pref = [[None, None], [None, None]]
    resid = [[None, None], [None, None]]
    for c in range(2):
        b0, r0, b1, r1 = _cum_select2(
            accv, bsum, c * 2048, 2048, jnp.int32(K0), jnp.int32(K1)
        )
        pref[0][c], resid[0][c] = b0, r0
        pref[1][c], resid[1][c] = b1, r1
    eq = jnp.logical_and(pref[0][0] == pref[1][0], pref[0][1] == pref[1][1])
    cp.wait()
    ones = jnp.ones((16,), jnp.int32)

    prefv = [
        [jnp.broadcast_to(pref[k][c], (16,)) for c in range(2)]
        for k in range(2)
    ]

    def scan(dual):
        @plsc.parallel_loop(0, CHUNK // 64, unroll=4)
        def _(t):
            base = t * 64
            cc = (t >> 1) & 1
            p0 = jnp.where(cc == 0, prefv[0][0], prefv[0][1])
            p1 = jnp.where(cc == 0, prefv[1][0], prefv[1][1])
            coff = cc * 2048
            for jj in range(4):
                v = buf[pl.ds(base + jj * 16, 16)]
                key = _key16(v)
                b1v = (key >> 21) + 1024
                idx = ((key >> 10) & 0x7FF) + coff
                plsc.addupdate_scatter(hist, [idx], ones, mask=b1v == p0)
                if dual:
                    plsc.addupdate_scatter(
                        hist, [idx + 4096], ones, mask=b1v == p1
                    )

    @pl.when(eq)
    def _():
        scan(False)

    @pl.when(jnp.logical_not(eq))
    def _():
        scan(True)

    _pass_epilogue(outh, hist, shared, tmp, red, sem, H2)
    _write_selvec(outsv, svm, pref, resid)


@functools.partial(
    pl.kernel,
    out_type=(
        jax.ShapeDtypeStruct((NC, H3), jnp.int32),
        jax.ShapeDtypeStruct((3, 16), jnp.int32),
    ),
    mesh=_mesh,
    compiler_params=_cparams,
    scratch_types=_pass_scratch(H3) + [
        pltpu.VMEM((NC, H2), jnp.int32),
        pltpu.VMEM((3, 16), jnp.int32),
        pltpu.VMEM((3, 16), jnp.int32),
        pltpu.VMEM((128,), jnp.int32),
    ],
)
def _pass3(data, h2, sv1, outh, outsv, buf, hist, shared, tmp, red, sem,
           accv, svin, svm, bsum):
    _, _, wid = _ids()
    cp = pltpu.async_copy(data.at[pl.ds(wid * CHUNK, CHUNK)], buf, sem)
    pltpu.sync_copy(h2, accv)
    pltpu.sync_copy(sv1, svin)
    _zero_flat(hist, H3)
    b1s = [[_pick(svin[k], c) for c in range(2)] for k in range(2)]
    rins = [[_pick(svin[2], 2 * c + k) for c in range(2)] for k in range(2)]
    eq2 = jnp.logical_and(b1s[0][0] == b1s[1][0], b1s[0][1] == b1s[1][1])
    pref = [[None, None], [None, None]]
    resid = [[None, None], [None, None]]
    for k in range(2):
        for c in range(2):
            kbase = k * 4096 if k == 0 else jnp.where(eq2, 0, 4096)
            b2, r2, _, _ = _cum_select2(
                accv, bsum, kbase + c * 2048, 2048, rins[k][c], rins[k][c]
            )
            pref[k][c] = (((b1s[k][c] + 1024) & 0x7FF) << 11) | b2
            resid[k][c] = r2
    eq3 = jnp.logical_and(pref[0][0] == pref[1][0], pref[0][1] == pref[1][1])
    cp.wait()
    ones = jnp.ones((16,), jnp.int32)

    prefv = [
        [jnp.broadcast_to(pref[k][c], (16,)) for c in range(2)]
        for k in range(2)
    ]

    def scan(dual):
        @plsc.parallel_loop(0, CHUNK // 64, unroll=4)
        def _(t):
            base = t * 64
            cc = (t >> 1) & 1
            p0 = jnp.where(cc == 0, prefv[0][0], prefv[0][1])
            p1 = jnp.where(cc == 0, prefv[1][0], prefv[1][1])
            coff = cc * 1024
            for jj in range(4):
                v = buf[pl.ds(base + jj * 16, 16)]
                key = _key16(v)
                p22 = (key >> 10) & 0x3FFFFF
                idx = (key & 0x3FF) + coff
                plsc.addupdate_scatter(hist, [idx], ones, mask=p22 == p0)
                if dual:
                    plsc.addupdate_scatter(
                        hist, [idx + 2048], ones, mask=p22 == p1
                    )

    @pl.when(eq3)
    def _():
        scan(False)

    @pl.when(jnp.logical_not(eq3))
    def _():
        scan(True)

    _pass_epilogue(outh, hist, shared, tmp, red, sem, H3)
    _write_selvec(outsv, svm, pref, resid)


@functools.partial(
    pl.kernel,
    out_type=jax.ShapeDtypeStruct((16,), jnp.float32),
    mesh=_mesh,
    compiler_params=_cparams,
    scratch_types=[
        pltpu.VMEM((NC, H3), jnp.int32),
        pltpu.VMEM((3, 16), jnp.int32),
        pltpu.VMEM((16,), jnp.float32),
        pltpu.VMEM((128,), jnp.int32),
    ],
)
def _sel3(hin, sel, out, acc, svin, ovm, bsum):
    _, _, wid = _ids()

    @pl.when(wid == 0)
    def _():
        pltpu.sync_copy(hin, acc)
        pltpu.sync_copy(sel, svin)
        lane = _lane_iota()
        p22s = [[_pick(svin[k], c) for c in range(2)] for k in range(2)]
        rins = [[_pick(svin[2], 2 * c + k) for c in range(2)] for k in range(2)]
        eq3 = jnp.logical_and(
            p22s[0][0] == p22s[1][0], p22s[0][1] == p22s[1][1]
        )
        meds = []
        for c in range(2):
            vals = []
            for k in range(2):
                kbase = k * 2048 if k == 0 else jnp.where(eq3, 0, 2048)
                b3, _, _, _ = _cum_select2(
                    acc, bsum, kbase + c * 1024, 1024, rins[k][c], rins[k][c]
                )
                key = (p22s[k][c] << 10) | b3
                u = key ^ ((key >> 31) & M31)
                vals.append(lax.bitcast_convert_type(u, jnp.float32))
            meds.append((vals[0] + vals[1]) * jnp.float32(0.5))
        zv = jnp.zeros((16,), jnp.float32)
        res = jnp.where(lane == 0, meds[0], zv)
        res = jnp.where(lane == 1, meds[1], res)
        ovm[pl.ds(0, 16)] = res
        pltpu.sync_copy(ovm, out)


def kernel(inputs):
    flat = inputs.reshape(8192, 128, 2).transpose(0, 2, 1).reshape(-1)
    h1 = _pass1(flat)
    h2, sv1 = _pass2(flat, h1)
    h3, sv2 = _pass3(flat, h2, sv1)
    med = _sel3(h3, sv2)
    return med[:2]


# speedup vs baseline: 1.0222x; 1.0150x over previous
import functools

import jax
import jax.numpy as jnp
from jax import lax
from jax.experimental import pallas as pl
from jax.experimental.pallas import tpu as pltpu
from jax.experimental.pallas import tpu_sc as plsc

N = 1048576
TOT = 2 * N
NC = 2
NS = 16
NW = NC * NS
CHUNK = TOT // NW
K0 = N // 2 - 1
K1 = N // 2

H1 = 4096
H2 = 8192
H3 = 4096

M31 = 0x7FFFFFFF

_mesh = plsc.VectorSubcoreMesh(core_axis_name="c", subcore_axis_name="s")
_cparams = pltpu.CompilerParams(needs_layout_passes=False)


def _ids():
    cid = lax.axis_index("c")
    sid = lax.axis_index("s")
    return cid, sid, sid * NC + cid


def _key16(v):
    u = plsc.bitcast(v, jnp.int32)
    return u ^ ((u >> 31) & M31)


def _lane_iota():
    return lax.iota(jnp.int32, 16)


def _pick(vec, i):
    return jnp.sum(jnp.where(_lane_iota() == i, vec, jnp.int32(0)))


def _zero_flat(ref, n):
    z = jnp.zeros((16,), jnp.int32)

    @plsc.parallel_loop(0, n // 16)
    def _(j):
        ref[pl.ds(j * 16, 16)] = z


def _cum_select2(acc, base, nb, ka, kb):
    def body(i, carry):
        run, bca, rsa, bcb, rsb = carry
        s = pl.ds(base + i * 16, 16)
        v = acc[0, s] + acc[1, s]
        cs = plsc.cumsum(v) + run
        ma = cs <= ka
        mb = cs <= kb
        one = jnp.int32(1)
        zero = jnp.int32(0)
        bca = bca + jnp.where(ma, one, zero)
        rsa = rsa + jnp.where(ma, v, zero)
        bcb = bcb + jnp.where(mb, one, zero)
        rsb = rsb + jnp.where(mb, v, zero)
        run = run + jnp.sum(v)
        return run, bca, rsa, bcb, rsb

    z = jnp.zeros((16,), jnp.int32)
    _, bca, rsa, bcb, rsb = lax.fori_loop(
        0, nb // 16, body, (jnp.int32(0), z, z, z, z)
    )
    return jnp.sum(bca), ka - jnp.sum(rsa), jnp.sum(bcb), kb - jnp.sum(rsb)


def _pass_epilogue(out, hist, shared, tmp, red, sem, h):
    cid, sid, _ = _ids()
    w = h // 16
    pltpu.sync_copy(hist, shared.at[sid])
    plsc.subcore_barrier()
    base = sid * w
    pltpu.sync_copy(shared.at[:, pl.ds(base, w)], tmp)

    @plsc.parallel_loop(0, w // 16)
    def _(j):
        s = pl.ds(j * 16, 16)
        acc = tmp[0, s]
        for r in range(1, 16):
            acc = acc + tmp[r, s]
        red[s] = acc

    pltpu.sync_copy(red, out.at[cid, pl.ds(base, w)])


def _pass_scratch(h):
    return [
        pltpu.VMEM((CHUNK,), jnp.float32),
        pltpu.VMEM((h,), jnp.int32),
        pltpu.VMEM_SHARED((16, h), jnp.int32),
        pltpu.VMEM((16, h // 16), jnp.int32),
        pltpu.VMEM((h // 16,), jnp.int32),
        pltpu.SemaphoreType.DMA,
    ]


def _write_selvec(outsv, svm, pref, resid):
    _, _, wid = _ids()

    @pl.when(wid == 0)
    def _():
        lane = _lane_iota()
        par = lane & 1
        svm[0] = jnp.where(par == 0, pref[0][0], pref[0][1])
        svm[1] = jnp.where(par == 0, pref[1][0], pref[1][1])
        resids = jnp.zeros((16,), jnp.int32)
        for c in range(2):
            for k in range(2):
                resids = jnp.where(lane == 2 * c + k, resid[k][c], resids)
        svm[2] = resids
        pltpu.sync_copy(svm, outsv)


@functools.partial(
    pl.kernel,
    out_type=jax.ShapeDtypeStruct((NC, H1), jnp.int32),
    mesh=_mesh,
    compiler_params=_cparams,
    scratch_types=_pass_scratch(H1),
)
def _pass1(data, out, buf, hist, shared, tmp, red, sem):
    _, _, wid = _ids()
    cp = pltpu.async_copy(data.at[pl.ds(wid * CHUNK, CHUNK)], buf, sem)
    _zero_flat(hist, H1)
    cp.wait()
    ones = jnp.ones((16,), jnp.int32)

    @plsc.parallel_loop(0, CHUNK // 64, unroll=4)
    def _(t):
        base = t * 64
        coff = ((t >> 1) & 1) * 2048 + 1024
        for jj in range(4):
            v = buf[pl.ds(base + jj * 16, 16)]
            key = _key16(v)
            idx = (key >> 21) + coff
            plsc.addupdate_scatter(hist, [idx], ones)

    _pass_epilogue(out, hist, shared, tmp, red, sem, H1)


@functools.partial(
    pl.kernel,
    out_type=(
        jax.ShapeDtypeStruct((NC, H2), jnp.int32),
        jax.ShapeDtypeStruct((3, 16), jnp.int32),
    ),
    mesh=_mesh,
    compiler_params=_cparams,
    scratch_types=_pass_scratch(H2) + [
        pltpu.VMEM((NC, H1), jnp.int32),
        pltpu.VMEM((3, 16), jnp.int32),
    ],
)
def _pass2(data, h1, outh, outsv, buf, hist, shared, tmp, red, sem, accv, svm):
    _, _, wid = _ids()
    cp = pltpu.async_copy(data.at[pl.ds(wid * CHUNK, CHUNK)], buf, sem)
    pltpu.sync_copy(h1, accv)
    _zero_flat(hist, H2)
    pref = [[None, None], [None, None]]
    resid = [[None, None], [None, None]]
    for c in range(2):
        b0, r0, b1, r1 = _cum_select2(
            accv, c * 2048, 2048, jnp.int32(K0), jnp.int32(K1)
        )
        pref[0][c], resid[0][c] = b0, r0
        pref[1][c], resid[1][c] = b1, r1
    eq = jnp.logical_and(pref[0][0] == pref[1][0], pref[0][1] == pref[1][1])
    cp.wait()
    ones = jnp.ones((16,), jnp.int32)

    prefv = [
        [jnp.broadcast_to(pref[k][c], (16,)) for c in range(2)]
        for k in range(2)
    ]

    def scan(dual):
        @plsc.parallel_loop(0, CHUNK // 64, unroll=4)
        def _(t):
            base = t * 64
            cc = (t >> 1) & 1
            p0 = jnp.where(cc == 0, prefv[0][0], prefv[0][1])
            p1 = jnp.where(cc == 0, prefv[1][0], prefv[1][1])
            coff = cc * 2048
            for jj in range(4):
                v = buf[pl.ds(base + jj * 16, 16)]
                key = _key16(v)
                b1v = (key >> 21) + 1024
                idx = ((key >> 10) & 0x7FF) + coff
                plsc.addupdate_scatter(hist, [idx], ones, mask=b1v == p0)
                if dual:
                    plsc.addupdate_scatter(
                        hist, [idx + 4096], ones, mask=b1v == p1
                    )

    @pl.when(eq)
    def _():
        scan(False)

    @pl.when(jnp.logical_not(eq))
    def _():
        scan(True)

    _pass_epilogue(outh, hist, shared, tmp, red, sem, H2)
    _write_selvec(outsv, svm, pref, resid)


@functools.partial(
    pl.kernel,
    out_type=(
        jax.ShapeDtypeStruct((NC, H3), jnp.int32),
        jax.ShapeDtypeStruct((3, 16), jnp.int32),
    ),
    mesh=_mesh,
    compiler_params=_cparams,
    scratch_types=_pass_scratch(H3) + [
        pltpu.VMEM((NC, H2), jnp.int32),
        pltpu.VMEM((3, 16), jnp.int32),
        pltpu.VMEM((3, 16), jnp.int32),
    ],
)
def _pass3(data, h2, sv1, outh, outsv, buf, hist, shared, tmp, red, sem,
           accv, svin, svm):
    _, _, wid = _ids()
    cp = pltpu.async_copy(data.at[pl.ds(wid * CHUNK, CHUNK)], buf, sem)
    pltpu.sync_copy(h2, accv)
    pltpu.sync_copy(sv1, svin)
    _zero_flat(hist, H3)
    b1s = [[_pick(svin[k], c) for c in range(2)] for k in range(2)]
    rins = [[_pick(svin[2], 2 * c + k) for c in range(2)] for k in range(2)]
    eq2 = jnp.logical_and(b1s[0][0] == b1s[1][0], b1s[0][1] == b1s[1][1])
    pref = [[None, None], [None, None]]
    resid = [[None, None], [None, None]]
    for k in range(2):
        for c in range(2):
            kbase = k * 4096 if k == 0 else jnp.where(eq2, 0, 4096)
            b2, r2, _, _ = _cum_select2(
                accv, kbase + c * 2048, 2048, rins[k][c], rins[k][c]
            )
            pref[k][c] = (((b1s[k][c] + 1024) & 0x7FF) << 11) | b2
            resid[k][c] = r2
    eq3 = jnp.logical_and(pref[0][0] == pref[1][0], pref[0][1] == pref[1][1])
    cp.wait()
    ones = jnp.ones((16,), jnp.int32)

    prefv = [
        [jnp.broadcast_to(pref[k][c], (16,)) for c in range(2)]
        for k in range(2)
    ]

    def scan(dual):
        @plsc.parallel_loop(0, CHUNK // 64, unroll=4)
        def _(t):
            base = t * 64
            cc = (t >> 1) & 1
            p0 = jnp.where(cc == 0, prefv[0][0], prefv[0][1])
            p1 = jnp.where(cc == 0, prefv[1][0], prefv[1][1])
            coff = cc * 1024
            for jj in range(4):
                v = buf[pl.ds(base + jj * 16, 16)]
                key = _key16(v)
                p22 = (key >> 10) & 0x3FFFFF
                idx = (key & 0x3FF) + coff
                plsc.addupdate_scatter(hist, [idx], ones, mask=p22 == p0)
                if dual:
                    plsc.addupdate_scatter(
                        hist, [idx + 2048], ones, mask=p22 == p1
                    )

    @pl.when(eq3)
    def _():
        scan(False)

    @pl.when(jnp.logical_not(eq3))
    def _():
        scan(True)

    _pass_epilogue(outh, hist, shared, tmp, red, sem, H3)
    _write_selvec(outsv, svm, pref, resid)


@functools.partial(
    pl.kernel,
    out_type=jax.ShapeDtypeStruct((16,), jnp.float32),
    mesh=_mesh,
    compiler_params=_cparams,
    scratch_types=[
        pltpu.VMEM((NC, H3), jnp.int32),
        pltpu.VMEM((3, 16), jnp.int32),
        pltpu.VMEM((16,), jnp.float32),
    ],
)
def _sel3(hin, sel, out, acc, svin, ovm):
    _, _, wid = _ids()

    @pl.when(wid == 0)
    def _():
        pltpu.sync_copy(hin, acc)
        pltpu.sync_copy(sel, svin)
        lane = _lane_iota()
        p22s = [[_pick(svin[k], c) for c in range(2)] for k in range(2)]
        rins = [[_pick(svin[2], 2 * c + k) for c in range(2)] for k in range(2)]
        eq3 = jnp.logical_and(
            p22s[0][0] == p22s[1][0], p22s[0][1] == p22s[1][1]
        )
        meds = []
        for c in range(2):
            vals = []
            for k in range(2):
                kbase = k * 2048 if k == 0 else jnp.where(eq3, 0, 2048)
                b3, _, _, _ = _cum_select2(
                    acc, kbase + c * 1024, 1024, rins[k][c], rins[k][c]
                )
                key = (p22s[k][c] << 10) | b3
                u = key ^ ((key >> 31) & M31)
                vals.append(lax.bitcast_convert_type(u, jnp.float32))
            meds.append((vals[0] + vals[1]) * jnp.float32(0.5))
        zv = jnp.zeros((16,), jnp.float32)
        res = jnp.where(lane == 0, meds[0], zv)
        res = jnp.where(lane == 1, meds[1], res)
        ovm[pl.ds(0, 16)] = res
        pltpu.sync_copy(ovm, out)


def kernel(inputs):
    flat = inputs.reshape(8192, 128, 2).transpose(0, 2, 1).reshape(-1)
    h1 = _pass1(flat)
    h2, sv1 = _pass2(flat, h1)
    h3, sv2 = _pass3(flat, h2, sv1)
    med = _sel3(h3, sv2)
    return med[:2]
